# baseline (speedup 1.0000x reference)
import functools

import jax
import jax.numpy as jnp
from jax import lax
from jax.experimental import pallas as pl
from jax.experimental.pallas import tpu as pltpu
from jax.experimental.pallas import tpu_sc as plsc

_VOCAB = 100000
_D = 64
_H = 64
_NCAND = 32
_EP = 10
_B = 4096

_NC = 2
_NS = 16
_NW = _NC * _NS
_RW = _B // _NW
_CH = 16
_NCHUNK = _RW // _CH
_L = 16


def _lane_iota():
    return lax.iota(jnp.int32, _L)


def _butterfly_sum(v):
    idx = _lane_iota()
    for dist in (8, 4, 2, 1):
        v = v + jnp.take(v, idx ^ dist)
    return v


def _build_sample_step():
    mesh = plsc.VectorSubcoreMesh(core_axis_name="c", subcore_axis_name="s")

    @functools.partial(
        pl.kernel,
        mesh=mesh,
        compiler_params=pltpu.CompilerParams(use_tc_tiling_on_sc=False,
                                             needs_layout_passes=False),
        out_type=[
            jax.ShapeDtypeStruct((_B,), jnp.int32),
            jax.ShapeDtypeStruct((_B, _D), jnp.float32),
            jax.ShapeDtypeStruct((_B, _D), jnp.float32),
        ],
        scratch_types=[
            pltpu.VMEM((_RW,), jnp.int32),
            pltpu.VMEM((_RW, _NCAND), jnp.int32),
            pltpu.VMEM((_RW, _NCAND), jnp.int32),
            pltpu.VMEM((_RW, _D), jnp.float32),
            pltpu.VMEM((_RW, _NCAND), jnp.float32),
            pltpu.VMEM((16, _D), jnp.float32),
            pltpu.VMEM((_CH, _NCAND, _D), jnp.float32),
            pltpu.VMEM((_CH, _NCAND, _D), jnp.float32),
            pltpu.VMEM((_RW,), jnp.int32),
            pltpu.VMEM((_RW, _D), jnp.float32),
            pltpu.VMEM((_RW, _D), jnp.float32),
            pltpu.SemaphoreType.DMA,
            pltpu.SemaphoreType.DMA,
            pltpu.SemaphoreType.DMA,
        ],
    )
    def sample_step(emb_hbm, adjn_hbm, adjet_hbm, edge_hbm, cur_hbm, x_hbm,
                    noise_hbm,
                    next_out, nxe_out, edg_out,
                    cur_v, adjn_v, adjet_v, x_v, noise_v, edge_v, cand_a,
                    cand_b, next_v, nxe_v, edg_v, sem, sem2, semb):
        wid = lax.axis_index("s") * _NC + lax.axis_index("c")
        base = wid * _RW
        pltpu.sync_copy(cur_hbm.at[pl.ds(base, _RW)], cur_v)
        cpn = pltpu.async_copy(adjn_hbm.at[cur_v], adjn_v, sem)
        cpe = pltpu.async_copy(adjet_hbm.at[cur_v], adjet_v, sem2)
        pltpu.sync_copy(x_hbm.at[pl.ds(base, _RW)], x_v)
        pltpu.sync_copy(noise_hbm.at[pl.ds(base, _RW)], noise_v)
        pltpu.sync_copy(edge_hbm, edge_v)
        cpn.wait()
        cpe.wait()

        lanes = _lane_iota()
        lane0 = lanes == 0

        def fire(ci, buf, s):
            row0 = ci * _CH
            for r in range(_CH):
                pltpu.async_copy(emb_hbm.at[adjn_v.at[row0 + r]],
                                 buf.at[r], s)

        def drain(buf, s):
            for r in range(_CH):
                pltpu.make_async_copy(emb_hbm.at[adjn_v.at[r]],
                                      buf.at[r], s).wait()

        def process_chunk(ci, cand_v):
            row0 = ci * _CH

            def row_body(r2, carry2):
                row = row0 + r2
                r2v = jnp.full((_L,), 0, jnp.int32) + r2
                rowv = jnp.full((_L,), 0, jnp.int32) + row
                xk = [x_v[row, pl.ds(k * _L, _L)] for k in range(4)]

                def cand_group(gi, carry3):
                    lo, hi = carry3
                    for j in range(4):
                        n = gi * 4 + j
                        p0 = cand_v[r2, n, pl.ds(0, _L)] * xk[0]
                        p1 = cand_v[r2, n, pl.ds(_L, _L)] * xk[1]
                        p2 = cand_v[r2, n, pl.ds(2 * _L, _L)] * xk[2]
                        p3 = cand_v[r2, n, pl.ds(3 * _L, _L)] * xk[3]
                        acc = _butterfly_sum((p0 + p2) + (p1 + p3))
                        lo = jnp.where((lanes == n), acc, lo)
                        hi = jnp.where((lanes == n - _L), acc, hi)
                    return lo, hi

                logits_lo, logits_hi = lax.fori_loop(
                    0, _NCAND // 4, cand_group,
                    (jnp.zeros((_L,), jnp.float32),
                     jnp.zeros((_L,), jnp.float32)), unroll=False)
                v_lo = noise_v[row, pl.ds(0, _L)] + logits_lo
                v_hi = noise_v[row, pl.ds(_L, _L)] + logits_hi
                m = jnp.maximum(jnp.max(v_lo), jnp.max(v_hi))
                eq_lo = v_lo == m
                eq_hi = v_hi == m
                any_lo = plsc.all_reduce_population_count(eq_lo) > 0
                ffs_lo = plsc.all_reduce_ffs(eq_lo)
                ffs_hi = plsc.all_reduce_ffs(eq_hi)
                tmp = jnp.where(any_lo, ffs_lo, ffs_hi + _L)
                nxt = plsc.load_gather(adjn_v, [rowv, tmp])
                et = plsc.load_gather(adjet_v, [rowv, tmp])
                plsc.store_scatter(next_v, [rowv], nxt, mask=lane0)
                for k in range(4):
                    dk = lanes + (k * _L)
                    emb = plsc.load_gather(cand_v, [r2v, tmp, dk])
                    edg = plsc.load_gather(edge_v, [et, dk])
                    nxe_v[row, pl.ds(k * _L, _L)] = emb
                    edg_v[row, pl.ds(k * _L, _L)] = edg
                return carry2

            lax.fori_loop(0, _CH, row_body, 0, unroll=2)

        fire(0, cand_a, sem)

        def outer_body(cj, carry):
            ci0 = 2 * cj
            fire(ci0 + 1, cand_b, semb)
            drain(cand_a, sem)
            process_chunk(ci0, cand_a)

            @pl.when(cj < _NCHUNK // 2 - 1)
            def _():
                fire(ci0 + 2, cand_a, sem)

            drain(cand_b, semb)
            process_chunk(ci0 + 1, cand_b)
            return carry

        lax.fori_loop(0, _NCHUNK // 2, outer_body, 0, unroll=False)
        pltpu.sync_copy(next_v, next_out.at[pl.ds(base, _RW)])
        pltpu.sync_copy(nxe_v, nxe_out.at[pl.ds(base, _RW)])
        pltpu.sync_copy(edg_v, edg_out.at[pl.ds(base, _RW)])

    return sample_step


_SAMPLE_STEP = _build_sample_step()


def _tc_body(edge_r, nxe_r, query_r, h_r, c_r, wihT_r, whhT_r, bih_r, bhh_r,
             f1T_r, f1b_r, f2T_r, f2b_r, h_o, c_o, x_o):
    input_ = jnp.concatenate([edge_r[...], nxe_r[...], query_r[...]], axis=-1)
    gates = (jnp.dot(input_, wihT_r[...]) + bih_r[...]
             + jnp.dot(h_r[...], whhT_r[...]) + bhh_r[...])
    i, f, g, o = jnp.split(gates, 4, axis=-1)
    i = jax.nn.sigmoid(i)
    f = jax.nn.sigmoid(f)
    g = jnp.tanh(g)
    o = jax.nn.sigmoid(o)
    c2 = f * c_r[...] + i * g
    h2 = o * jnp.tanh(c2)
    h_o[...] = h2
    c_o[...] = c2
    x_o[...] = (jnp.dot(jax.nn.relu(jnp.dot(h2, f1T_r[...]) + f1b_r[...]),
                        f2T_r[...]) + f2b_r[...])


_TC_STEP = pl.pallas_call(
    _tc_body,
    out_shape=(
        jax.ShapeDtypeStruct((_B, _H), jnp.float32),
        jax.ShapeDtypeStruct((_B, _H), jnp.float32),
        jax.ShapeDtypeStruct((_B, _D), jnp.float32),
    ),
)


def kernel(start_inds, end_inds, embeddings, edge_embeds, W_ih, W_hh,
           b_ih, b_hh, fc1_W, fc1_b, fc2_W, fc2_b, adj_nodes, adj_etypes):
    start_embeds = jnp.take(embeddings, start_inds, axis=0)
    end_embeds = jnp.take(embeddings, end_inds, axis=0)
    query = jnp.concatenate([start_embeds, end_embeds], axis=-1)
    h = jnp.zeros((_B, _H), dtype=jnp.float32)
    c = jnp.zeros((_B, _H), dtype=jnp.float32)
    base_key = jax.random.key(42)
    noise = [jax.random.gumbel(jax.random.fold_in(base_key, s),
                               (_B, _NCAND), jnp.float32)
             for s in range(_EP - 1)]
    wihT = W_ih.T
    whhT = W_hh.T
    f1T = fc1_W.T
    f2T = fc2_W.T
    bih = b_ih.reshape(1, -1)
    bhh = b_hh.reshape(1, -1)
    f1b = fc1_b.reshape(1, -1)
    f2b = fc2_b.reshape(1, -1)

    edge_e = jnp.zeros((_B, _D), dtype=jnp.float32)
    nxe = start_embeds
    current = start_inds
    out_embeds = [start_embeds]
    out_inds = [start_inds.astype(jnp.int32)]
    for step in range(_EP - 1):
        h, c, x = _TC_STEP(edge_e, nxe, query, h, c, wihT, whhT, bih, bhh,
                           f1T, f1b, f2T, f2b)
        current, nxe, edge_e = _SAMPLE_STEP(
            embeddings, adj_nodes, adj_etypes, edge_embeds, current, x,
            noise[step])
        out_embeds.append(nxe)
        out_inds.append(current)
    return (jnp.stack(out_embeds, axis=0), jnp.stack(out_inds, axis=0),
            start_embeds, end_embeds)


# speedup vs baseline: 1.0280x; 1.0280x over previous
import functools

import jax
import jax.numpy as jnp
from jax import lax
from jax.experimental import pallas as pl
from jax.experimental.pallas import tpu as pltpu
from jax.experimental.pallas import tpu_sc as plsc

_VOCAB = 100000
_D = 64
_H = 64
_NCAND = 32
_EP = 10
_B = 4096

_NC = 2
_NS = 16
_NW = _NC * _NS
_RW = _B // _NW
_CH = 16
_NCHUNK = _RW // _CH
_L = 16


def _lane_iota():
    return lax.iota(jnp.int32, _L)


def _butterfly_sum(v):
    idx = _lane_iota()
    for dist in (8, 4, 2, 1):
        v = v + jnp.take(v, idx ^ dist)
    return v


def _build_sample_step():
    mesh = plsc.VectorSubcoreMesh(core_axis_name="c", subcore_axis_name="s")

    @functools.partial(
        pl.kernel,
        mesh=mesh,
        compiler_params=pltpu.CompilerParams(use_tc_tiling_on_sc=False,
                                             needs_layout_passes=False),
        out_type=[
            jax.ShapeDtypeStruct((_B,), jnp.int32),
            jax.ShapeDtypeStruct((_B, _D), jnp.float32),
            jax.ShapeDtypeStruct((_B, _D), jnp.float32),
        ],
        scratch_types=[
            pltpu.VMEM((_RW,), jnp.int32),
            pltpu.VMEM((_RW, _NCAND), jnp.int32),
            pltpu.VMEM((_RW, _NCAND), jnp.int32),
            pltpu.VMEM((_RW, _D), jnp.float32),
            pltpu.VMEM((_RW, _NCAND), jnp.float32),
            pltpu.VMEM((16, _D), jnp.float32),
            pltpu.VMEM((_CH, _NCAND, _D), jnp.float32),
            pltpu.VMEM((_CH, _NCAND, _D), jnp.float32),
            pltpu.VMEM((_RW,), jnp.int32),
            pltpu.VMEM((_RW, _D), jnp.float32),
            pltpu.VMEM((_RW, _D), jnp.float32),
            pltpu.SemaphoreType.DMA,
            pltpu.SemaphoreType.DMA,
            pltpu.SemaphoreType.DMA,
        ],
    )
    def sample_step(emb_hbm, adjn_hbm, adjet_hbm, edge_hbm, cur_hbm, x_hbm,
                    noise_hbm,
                    next_out, nxe_out, edg_out,
                    cur_v, adjn_v, adjet_v, x_v, noise_v, edge_v, cand_a,
                    cand_b, next_v, nxe_v, edg_v, sem, sem2, semb):
        wid = lax.axis_index("s") * _NC + lax.axis_index("c")
        base = wid * _RW
        pltpu.sync_copy(cur_hbm.at[pl.ds(base, _RW)], cur_v)
        cpn = pltpu.async_copy(adjn_hbm.at[cur_v], adjn_v, sem)
        cpe = pltpu.async_copy(adjet_hbm.at[cur_v], adjet_v, sem2)
        pltpu.sync_copy(x_hbm.at[pl.ds(base, _RW)], x_v)
        pltpu.sync_copy(noise_hbm.at[pl.ds(base, _RW)], noise_v)
        pltpu.sync_copy(edge_hbm, edge_v)
        cpn.wait()
        cpe.wait()

        lanes = _lane_iota()
        lane0 = lanes == 0

        def fire(ci, buf, s):
            row0 = ci * _CH
            for r in range(_CH):
                pltpu.async_copy(emb_hbm.at[adjn_v.at[row0 + r]],
                                 buf.at[r], s)

        def drain(buf, s):
            for r in range(_CH):
                pltpu.make_async_copy(emb_hbm.at[adjn_v.at[r]],
                                      buf.at[r], s).wait()

        def process_chunk(ci, cand_v):
            row0 = ci * _CH

            def row_body(r2, carry2):
                row = row0 + r2
                r2v = jnp.full((_L,), 0, jnp.int32) + r2
                rowv = jnp.full((_L,), 0, jnp.int32) + row
                xk = [x_v[row, pl.ds(k * _L, _L)] for k in range(4)]

                logits_lo = jnp.zeros((_L,), jnp.float32)
                logits_hi = jnp.zeros((_L,), jnp.float32)
                for n in range(_NCAND):
                    p0 = cand_v[r2, n, pl.ds(0, _L)] * xk[0]
                    p1 = cand_v[r2, n, pl.ds(_L, _L)] * xk[1]
                    p2 = cand_v[r2, n, pl.ds(2 * _L, _L)] * xk[2]
                    p3 = cand_v[r2, n, pl.ds(3 * _L, _L)] * xk[3]
                    acc = _butterfly_sum((p0 + p2) + (p1 + p3))
                    if n < _L:
                        logits_lo = jnp.where(lanes == n, acc, logits_lo)
                    else:
                        logits_hi = jnp.where(lanes == n - _L, acc, logits_hi)
                v_lo = noise_v[row, pl.ds(0, _L)] + logits_lo
                v_hi = noise_v[row, pl.ds(_L, _L)] + logits_hi
                m = jnp.maximum(jnp.max(v_lo), jnp.max(v_hi))
                eq_lo = v_lo == m
                eq_hi = v_hi == m
                any_lo = plsc.all_reduce_population_count(eq_lo) > 0
                ffs_lo = plsc.all_reduce_ffs(eq_lo)
                ffs_hi = plsc.all_reduce_ffs(eq_hi)
                tmp = jnp.where(any_lo, ffs_lo, ffs_hi + _L)
                nxt = plsc.load_gather(adjn_v, [rowv, tmp])
                et = plsc.load_gather(adjet_v, [rowv, tmp])
                plsc.store_scatter(next_v, [rowv], nxt, mask=lane0)
                for k in range(4):
                    dk = lanes + (k * _L)
                    emb = plsc.load_gather(cand_v, [r2v, tmp, dk])
                    edg = plsc.load_gather(edge_v, [et, dk])
                    nxe_v[row, pl.ds(k * _L, _L)] = emb
                    edg_v[row, pl.ds(k * _L, _L)] = edg
                return carry2

            lax.fori_loop(0, _CH, row_body, 0, unroll=2)

        fire(0, cand_a, sem)

        def outer_body(cj, carry):
            ci0 = 2 * cj
            fire(ci0 + 1, cand_b, semb)
            drain(cand_a, sem)
            process_chunk(ci0, cand_a)

            @pl.when(cj < _NCHUNK // 2 - 1)
            def _():
                fire(ci0 + 2, cand_a, sem)

            drain(cand_b, semb)
            process_chunk(ci0 + 1, cand_b)
            return carry

        lax.fori_loop(0, _NCHUNK // 2, outer_body, 0, unroll=False)
        pltpu.sync_copy(next_v, next_out.at[pl.ds(base, _RW)])
        pltpu.sync_copy(nxe_v, nxe_out.at[pl.ds(base, _RW)])
        pltpu.sync_copy(edg_v, edg_out.at[pl.ds(base, _RW)])

    return sample_step


_SAMPLE_STEP = _build_sample_step()


def _tc_body(edge_r, nxe_r, query_r, h_r, c_r, wihT_r, whhT_r, bih_r, bhh_r,
             f1T_r, f1b_r, f2T_r, f2b_r, h_o, c_o, x_o):
    input_ = jnp.concatenate([edge_r[...], nxe_r[...], query_r[...]], axis=-1)
    gates = (jnp.dot(input_, wihT_r[...]) + bih_r[...]
             + jnp.dot(h_r[...], whhT_r[...]) + bhh_r[...])
    i, f, g, o = jnp.split(gates, 4, axis=-1)
    i = jax.nn.sigmoid(i)
    f = jax.nn.sigmoid(f)
    g = jnp.tanh(g)
    o = jax.nn.sigmoid(o)
    c2 = f * c_r[...] + i * g
    h2 = o * jnp.tanh(c2)
    h_o[...] = h2
    c_o[...] = c2
    x_o[...] = (jnp.dot(jax.nn.relu(jnp.dot(h2, f1T_r[...]) + f1b_r[...]),
                        f2T_r[...]) + f2b_r[...])


_TC_STEP = pl.pallas_call(
    _tc_body,
    out_shape=(
        jax.ShapeDtypeStruct((_B, _H), jnp.float32),
        jax.ShapeDtypeStruct((_B, _H), jnp.float32),
        jax.ShapeDtypeStruct((_B, _D), jnp.float32),
    ),
)


def kernel(start_inds, end_inds, embeddings, edge_embeds, W_ih, W_hh,
           b_ih, b_hh, fc1_W, fc1_b, fc2_W, fc2_b, adj_nodes, adj_etypes):
    start_embeds = jnp.take(embeddings, start_inds, axis=0)
    end_embeds = jnp.take(embeddings, end_inds, axis=0)
    query = jnp.concatenate([start_embeds, end_embeds], axis=-1)
    h = jnp.zeros((_B, _H), dtype=jnp.float32)
    c = jnp.zeros((_B, _H), dtype=jnp.float32)
    base_key = jax.random.key(42)
    noise = [jax.random.gumbel(jax.random.fold_in(base_key, s),
                               (_B, _NCAND), jnp.float32)
             for s in range(_EP - 1)]
    wihT = W_ih.T
    whhT = W_hh.T
    f1T = fc1_W.T
    f2T = fc2_W.T
    bih = b_ih.reshape(1, -1)
    bhh = b_hh.reshape(1, -1)
    f1b = fc1_b.reshape(1, -1)
    f2b = fc2_b.reshape(1, -1)

    edge_e = jnp.zeros((_B, _D), dtype=jnp.float32)
    nxe = start_embeds
    current = start_inds
    out_embeds = [start_embeds]
    out_inds = [start_inds.astype(jnp.int32)]
    for step in range(_EP - 1):
        h, c, x = _TC_STEP(edge_e, nxe, query, h, c, wihT, whhT, bih, bhh,
                           f1T, f1b, f2T, f2b)
        current, nxe, edge_e = _SAMPLE_STEP(
            embeddings, adj_nodes, adj_etypes, edge_embeds, current, x,
            noise[step])
        out_embeds.append(nxe)
        out_inds.append(current)
    return (jnp.stack(out_embeds, axis=0), jnp.stack(out_inds, axis=0),
            start_embeds, end_embeds)


# speedup vs baseline: 1.0355x; 1.0073x over previous
import functools

import jax
import jax.numpy as jnp
from jax import lax
from jax.experimental import pallas as pl
from jax.experimental.pallas import tpu as pltpu
from jax.experimental.pallas import tpu_sc as plsc

_VOCAB = 100000
_D = 64
_H = 64
_NCAND = 32
_EP = 10
_B = 4096

_NC = 2
_NS = 16
_NW = _NC * _NS
_RW = _B // _NW
_CH = 16
_NCHUNK = _RW // _CH
_L = 16


def _lane_iota():
    return lax.iota(jnp.int32, _L)


def _butterfly_sum(v):
    idx = _lane_iota()
    for dist in (8, 4, 2, 1):
        v = v + jnp.take(v, idx ^ dist)
    return v


def _build_sample_step():
    mesh = plsc.VectorSubcoreMesh(core_axis_name="c", subcore_axis_name="s")

    @functools.partial(
        pl.kernel,
        mesh=mesh,
        compiler_params=pltpu.CompilerParams(use_tc_tiling_on_sc=False,
                                             needs_layout_passes=False),
        out_type=[
            jax.ShapeDtypeStruct((_B,), jnp.int32),
            jax.ShapeDtypeStruct((_B, _D), jnp.float32),
            jax.ShapeDtypeStruct((_B, _D), jnp.float32),
        ],
        scratch_types=[
            pltpu.VMEM((_RW,), jnp.int32),
            pltpu.VMEM((_RW, _NCAND), jnp.int32),
            pltpu.VMEM((_RW, _NCAND), jnp.int32),
            pltpu.VMEM((_RW, _D), jnp.float32),
            pltpu.VMEM((_RW, _NCAND), jnp.float32),
            pltpu.VMEM((16, _D), jnp.float32),
            pltpu.VMEM((_CH, _NCAND, _D), jnp.float32),
            pltpu.VMEM((_CH, _NCAND, _D), jnp.float32),
            pltpu.VMEM((_RW,), jnp.int32),
            pltpu.VMEM((_RW, _D), jnp.float32),
            pltpu.VMEM((_RW, _D), jnp.float32),
            pltpu.SemaphoreType.DMA,
            pltpu.SemaphoreType.DMA,
            pltpu.SemaphoreType.DMA,
        ],
    )
    def sample_step(emb_hbm, adjn_hbm, adjet_hbm, edge_hbm, cur_hbm, x_hbm,
                    noise_hbm,
                    next_out, nxe_out, edg_out,
                    cur_v, adjn_v, adjet_v, x_v, noise_v, edge_v, cand_a,
                    cand_b, next_v, nxe_v, edg_v, sem, sem2, semb):
        wid = lax.axis_index("s") * _NC + lax.axis_index("c")
        base = wid * _RW
        pltpu.sync_copy(cur_hbm.at[pl.ds(base, _RW)], cur_v)
        cpn = pltpu.async_copy(adjn_hbm.at[cur_v], adjn_v, sem)
        cpe = pltpu.async_copy(adjet_hbm.at[cur_v], adjet_v, sem2)
        pltpu.sync_copy(x_hbm.at[pl.ds(base, _RW)], x_v)
        pltpu.sync_copy(noise_hbm.at[pl.ds(base, _RW)], noise_v)
        pltpu.sync_copy(edge_hbm, edge_v)
        cpn.wait()
        cpe.wait()

        lanes = _lane_iota()
        lane0 = lanes == 0

        def fire(ci, buf, s):
            row0 = ci * _CH
            for r in range(_CH):
                pltpu.async_copy(emb_hbm.at[adjn_v.at[row0 + r]],
                                 buf.at[r], s)

        def drain(buf, s):
            for r in range(_CH):
                pltpu.make_async_copy(emb_hbm.at[adjn_v.at[r]],
                                      buf.at[r], s).wait()

        def process_chunk(ci, cand_v):
            row0 = ci * _CH

            def row_body(r2, carry2):
                row = row0 + r2
                r2v = jnp.full((_L,), 0, jnp.int32) + r2
                rowv = jnp.full((_L,), 0, jnp.int32) + row
                xk = [x_v[row, pl.ds(k * _L, _L)] for k in range(4)]

                los = [jnp.zeros((_L,), jnp.float32) for _ in range(4)]
                his = [jnp.zeros((_L,), jnp.float32) for _ in range(4)]
                for n in range(_NCAND):
                    p0 = cand_v[r2, n, pl.ds(0, _L)] * xk[0]
                    p1 = cand_v[r2, n, pl.ds(_L, _L)] * xk[1]
                    p2 = cand_v[r2, n, pl.ds(2 * _L, _L)] * xk[2]
                    p3 = cand_v[r2, n, pl.ds(3 * _L, _L)] * xk[3]
                    acc = _butterfly_sum((p0 + p2) + (p1 + p3))
                    if n < _L:
                        los[n % 4] = jnp.where(lanes == n, acc, los[n % 4])
                    else:
                        his[n % 4] = jnp.where(lanes == n - _L, acc,
                                               his[n % 4])
                logits_lo = (los[0] + los[1]) + (los[2] + los[3])
                logits_hi = (his[0] + his[1]) + (his[2] + his[3])
                v_lo = noise_v[row, pl.ds(0, _L)] + logits_lo
                v_hi = noise_v[row, pl.ds(_L, _L)] + logits_hi
                m = jnp.maximum(jnp.max(v_lo), jnp.max(v_hi))
                eq_lo = v_lo == m
                eq_hi = v_hi == m
                any_lo = plsc.all_reduce_population_count(eq_lo) > 0
                ffs_lo = plsc.all_reduce_ffs(eq_lo)
                ffs_hi = plsc.all_reduce_ffs(eq_hi)
                tmp = jnp.where(any_lo, ffs_lo, ffs_hi + _L)
                nxt = plsc.load_gather(adjn_v, [rowv, tmp])
                et = plsc.load_gather(adjet_v, [rowv, tmp])
                plsc.store_scatter(next_v, [rowv], nxt, mask=lane0)
                for k in range(4):
                    dk = lanes + (k * _L)
                    emb = plsc.load_gather(cand_v, [r2v, tmp, dk])
                    edg = plsc.load_gather(edge_v, [et, dk])
                    nxe_v[row, pl.ds(k * _L, _L)] = emb
                    edg_v[row, pl.ds(k * _L, _L)] = edg
                return carry2

            lax.fori_loop(0, _CH, row_body, 0, unroll=False)

        fire(0, cand_a, sem)

        def outer_body(cj, carry):
            ci0 = 2 * cj
            fire(ci0 + 1, cand_b, semb)
            drain(cand_a, sem)
            process_chunk(ci0, cand_a)

            @pl.when(cj < _NCHUNK // 2 - 1)
            def _():
                fire(ci0 + 2, cand_a, sem)

            drain(cand_b, semb)
            process_chunk(ci0 + 1, cand_b)
            return carry

        lax.fori_loop(0, _NCHUNK // 2, outer_body, 0, unroll=False)
        pltpu.sync_copy(next_v, next_out.at[pl.ds(base, _RW)])
        pltpu.sync_copy(nxe_v, nxe_out.at[pl.ds(base, _RW)])
        pltpu.sync_copy(edg_v, edg_out.at[pl.ds(base, _RW)])

    return sample_step


_SAMPLE_STEP = _build_sample_step()


def _tc_body(edge_r, nxe_r, query_r, h_r, c_r, wihT_r, whhT_r, bih_r, bhh_r,
             f1T_r, f1b_r, f2T_r, f2b_r, h_o, c_o, x_o):
    input_ = jnp.concatenate([edge_r[...], nxe_r[...], query_r[...]], axis=-1)
    gates = (jnp.dot(input_, wihT_r[...]) + bih_r[...]
             + jnp.dot(h_r[...], whhT_r[...]) + bhh_r[...])
    i, f, g, o = jnp.split(gates, 4, axis=-1)
    i = jax.nn.sigmoid(i)
    f = jax.nn.sigmoid(f)
    g = jnp.tanh(g)
    o = jax.nn.sigmoid(o)
    c2 = f * c_r[...] + i * g
    h2 = o * jnp.tanh(c2)
    h_o[...] = h2
    c_o[...] = c2
    x_o[...] = (jnp.dot(jax.nn.relu(jnp.dot(h2, f1T_r[...]) + f1b_r[...]),
                        f2T_r[...]) + f2b_r[...])


_TC_STEP = pl.pallas_call(
    _tc_body,
    out_shape=(
        jax.ShapeDtypeStruct((_B, _H), jnp.float32),
        jax.ShapeDtypeStruct((_B, _H), jnp.float32),
        jax.ShapeDtypeStruct((_B, _D), jnp.float32),
    ),
)


def kernel(start_inds, end_inds, embeddings, edge_embeds, W_ih, W_hh,
           b_ih, b_hh, fc1_W, fc1_b, fc2_W, fc2_b, adj_nodes, adj_etypes):
    start_embeds = jnp.take(embeddings, start_inds, axis=0)
    end_embeds = jnp.take(embeddings, end_inds, axis=0)
    query = jnp.concatenate([start_embeds, end_embeds], axis=-1)
    h = jnp.zeros((_B, _H), dtype=jnp.float32)
    c = jnp.zeros((_B, _H), dtype=jnp.float32)
    base_key = jax.random.key(42)
    noise = [jax.random.gumbel(jax.random.fold_in(base_key, s),
                               (_B, _NCAND), jnp.float32)
             for s in range(_EP - 1)]
    wihT = W_ih.T
    whhT = W_hh.T
    f1T = fc1_W.T
    f2T = fc2_W.T
    bih = b_ih.reshape(1, -1)
    bhh = b_hh.reshape(1, -1)
    f1b = fc1_b.reshape(1, -1)
    f2b = fc2_b.reshape(1, -1)

    edge_e = jnp.zeros((_B, _D), dtype=jnp.float32)
    nxe = start_embeds
    current = start_inds
    out_embeds = [start_embeds]
    out_inds = [start_inds.astype(jnp.int32)]
    for step in range(_EP - 1):
        h, c, x = _TC_STEP(edge_e, nxe, query, h, c, wihT, whhT, bih, bhh,
                           f1T, f1b, f2T, f2b)
        current, nxe, edge_e = _SAMPLE_STEP(
            embeddings, adj_nodes, adj_etypes, edge_embeds, current, x,
            noise[step])
        out_embeds.append(nxe)
        out_inds.append(current)
    return (jnp.stack(out_embeds, axis=0), jnp.stack(out_inds, axis=0),
            start_embeds, end_embeds)


# speedup vs baseline: 1.1423x; 1.1031x over previous
import functools

import jax
import jax.numpy as jnp
from jax import lax
from jax.experimental import pallas as pl
from jax.experimental.pallas import tpu as pltpu
from jax.experimental.pallas import tpu_sc as plsc

_VOCAB = 100000
_D = 64
_H = 64
_NCAND = 32
_EP = 10
_B = 4096

_NC = 2
_NS = 16
_NW = _NC * _NS
_RW = _B // _NW
_CH = 16
_NCHUNK = _RW // _CH
_L = 16


def _lane_iota():
    return lax.iota(jnp.int32, _L)


def _butterfly_sum(v):
    idx = _lane_iota()
    for dist in (8, 4, 2, 1):
        v = v + jnp.take(v, idx ^ dist)
    return v


def _build_sample_step():
    mesh = plsc.VectorSubcoreMesh(core_axis_name="c", subcore_axis_name="s")

    @functools.partial(
        pl.kernel,
        mesh=mesh,
        compiler_params=pltpu.CompilerParams(use_tc_tiling_on_sc=False,
                                             needs_layout_passes=False),
        out_type=[
            jax.ShapeDtypeStruct((_B,), jnp.int32),
            jax.ShapeDtypeStruct((_B, 2 * _D), jnp.float32),
        ],
        scratch_types=[
            pltpu.VMEM((_RW,), jnp.int32),
            pltpu.VMEM((_RW, _NCAND), jnp.int32),
            pltpu.VMEM((_RW, _NCAND), jnp.int32),
            pltpu.VMEM((_RW, 2 * _D), jnp.float32),
            pltpu.VMEM((_RW, _NCAND), jnp.float32),
            pltpu.VMEM((16, _D), jnp.float32),
            pltpu.VMEM((_CH, _NCAND, _D), jnp.float32),
            pltpu.VMEM((_CH, _NCAND, _D), jnp.float32),
            pltpu.VMEM((_RW,), jnp.int32),
            pltpu.VMEM((_RW, 2 * _D), jnp.float32),
            pltpu.SemaphoreType.DMA,
            pltpu.SemaphoreType.DMA,
            pltpu.SemaphoreType.DMA,
        ],
    )
    def sample_step(emb_hbm, adjn_hbm, adjet_hbm, edge_hbm, cur_hbm, x_hbm,
                    noise_hbm,
                    next_out, ea_out,
                    cur_v, adjn_v, adjet_v, x_v, noise_v, edge_v, cand_a,
                    cand_b, next_v, ea_v, sem, sem2, semb):
        wid = lax.axis_index("s") * _NC + lax.axis_index("c")
        base = wid * _RW
        pltpu.sync_copy(cur_hbm.at[pl.ds(base, _RW)], cur_v)
        cpn = pltpu.async_copy(adjn_hbm.at[cur_v], adjn_v, sem)
        cpe = pltpu.async_copy(adjet_hbm.at[cur_v], adjet_v, sem2)
        pltpu.sync_copy(x_hbm.at[pl.ds(base, _RW)], x_v)
        pltpu.sync_copy(noise_hbm.at[pl.ds(base, _RW)], noise_v)
        pltpu.sync_copy(edge_hbm, edge_v)
        cpn.wait()
        cpe.wait()

        lanes = _lane_iota()
        lane0 = lanes == 0

        def fire(ci, buf, s):
            row0 = ci * _CH
            for r in range(_CH):
                pltpu.async_copy(emb_hbm.at[adjn_v.at[row0 + r]],
                                 buf.at[r], s)

        def drain(buf, s):
            for r in range(_CH):
                pltpu.make_async_copy(emb_hbm.at[adjn_v.at[r]],
                                      buf.at[r], s).wait()

        def process_chunk(ci, cand_v):
            row0 = ci * _CH

            def row_body(r2, carry2):
                row = row0 + r2
                r2v = jnp.full((_L,), 0, jnp.int32) + r2
                rowv = jnp.full((_L,), 0, jnp.int32) + row
                xk = [x_v[row, pl.ds(k * _L, _L)] for k in range(4)]

                los = [jnp.zeros((_L,), jnp.float32) for _ in range(4)]
                his = [jnp.zeros((_L,), jnp.float32) for _ in range(4)]
                for n in range(_NCAND):
                    p0 = cand_v[r2, n, pl.ds(0, _L)] * xk[0]
                    p1 = cand_v[r2, n, pl.ds(_L, _L)] * xk[1]
                    p2 = cand_v[r2, n, pl.ds(2 * _L, _L)] * xk[2]
                    p3 = cand_v[r2, n, pl.ds(3 * _L, _L)] * xk[3]
                    acc = _butterfly_sum((p0 + p2) + (p1 + p3))
                    if n < _L:
                        los[n % 4] = jnp.where(lanes == n, acc, los[n % 4])
                    else:
                        his[n % 4] = jnp.where(lanes == n - _L, acc,
                                               his[n % 4])
                logits_lo = (los[0] + los[1]) + (los[2] + los[3])
                logits_hi = (his[0] + his[1]) + (his[2] + his[3])
                v_lo = noise_v[row, pl.ds(0, _L)] + logits_lo
                v_hi = noise_v[row, pl.ds(_L, _L)] + logits_hi
                m = jnp.maximum(jnp.max(v_lo), jnp.max(v_hi))
                eq_lo = v_lo == m
                eq_hi = v_hi == m
                any_lo = plsc.all_reduce_population_count(eq_lo) > 0
                ffs_lo = plsc.all_reduce_ffs(eq_lo)
                ffs_hi = plsc.all_reduce_ffs(eq_hi)
                tmp = jnp.where(any_lo, ffs_lo, ffs_hi + _L)
                nxt = plsc.load_gather(adjn_v, [rowv, tmp])
                et = plsc.load_gather(adjet_v, [rowv, tmp])
                plsc.store_scatter(next_v, [rowv], nxt, mask=lane0)
                for k in range(4):
                    dk = lanes + (k * _L)
                    emb = plsc.load_gather(cand_v, [r2v, tmp, dk])
                    edg = plsc.load_gather(edge_v, [et, dk])
                    ea_v[row, pl.ds(_D + k * _L, _L)] = emb
                    ea_v[row, pl.ds(k * _L, _L)] = edg
                return carry2

            lax.fori_loop(0, _CH, row_body, 0, unroll=False)

        fire(0, cand_a, sem)

        def outer_body(cj, carry):
            ci0 = 2 * cj
            fire(ci0 + 1, cand_b, semb)
            drain(cand_a, sem)
            process_chunk(ci0, cand_a)

            @pl.when(cj < _NCHUNK // 2 - 1)
            def _():
                fire(ci0 + 2, cand_a, sem)

            drain(cand_b, semb)
            process_chunk(ci0 + 1, cand_b)
            return carry

        lax.fori_loop(0, _NCHUNK // 2, outer_body, 0, unroll=False)
        pltpu.sync_copy(next_v, next_out.at[pl.ds(base, _RW)])
        pltpu.sync_copy(ea_v, ea_out.at[pl.ds(base, _RW)])

    return sample_step


_SAMPLE_STEP = _build_sample_step()


def _tc_body(ea_r, query_r, h_r, c_r, wihT_r, whhT_r, bih_r, bhh_r,
             f1T_r, f1b_r, f2T_r, f2b_r, h_o, c_o, x_o):
    input_ = jnp.concatenate([ea_r[...], query_r[...]], axis=-1)
    gates = (jnp.dot(input_, wihT_r[...]) + bih_r[...]
             + jnp.dot(h_r[...], whhT_r[...]) + bhh_r[...])
    i, f, g, o = jnp.split(gates, 4, axis=-1)
    i = jax.nn.sigmoid(i)
    f = jax.nn.sigmoid(f)
    g = jnp.tanh(g)
    o = jax.nn.sigmoid(o)
    c2 = f * c_r[...] + i * g
    h2 = o * jnp.tanh(c2)
    h_o[...] = h2
    c_o[...] = c2
    x = (jnp.dot(jax.nn.relu(jnp.dot(h2, f1T_r[...]) + f1b_r[...]),
                 f2T_r[...]) + f2b_r[...])
    x_o[...] = jnp.concatenate(
        [x, jnp.zeros((_B, _D), jnp.float32)], axis=-1)


_TC_STEP = pl.pallas_call(
    _tc_body,
    out_shape=(
        jax.ShapeDtypeStruct((_B, _H), jnp.float32),
        jax.ShapeDtypeStruct((_B, _H), jnp.float32),
        jax.ShapeDtypeStruct((_B, 2 * _D), jnp.float32),
    ),
)


def kernel(start_inds, end_inds, embeddings, edge_embeds, W_ih, W_hh,
           b_ih, b_hh, fc1_W, fc1_b, fc2_W, fc2_b, adj_nodes, adj_etypes):
    start_embeds = jnp.take(embeddings, start_inds, axis=0)
    end_embeds = jnp.take(embeddings, end_inds, axis=0)
    query = jnp.concatenate([start_embeds, end_embeds], axis=-1)
    h = jnp.zeros((_B, _H), dtype=jnp.float32)
    c = jnp.zeros((_B, _H), dtype=jnp.float32)
    base_key = jax.random.key(42)
    noise = [jax.random.gumbel(jax.random.fold_in(base_key, s),
                               (_B, _NCAND), jnp.float32)
             for s in range(_EP - 1)]
    wihT = W_ih.T
    whhT = W_hh.T
    f1T = fc1_W.T
    f2T = fc2_W.T
    bih = b_ih.reshape(1, -1)
    bhh = b_hh.reshape(1, -1)
    f1b = fc1_b.reshape(1, -1)
    f2b = fc2_b.reshape(1, -1)

    ea = jnp.concatenate([jnp.zeros((_B, _D), dtype=jnp.float32),
                          start_embeds], axis=-1)
    current = start_inds
    out_embeds = [start_embeds]
    out_inds = [start_inds.astype(jnp.int32)]
    for step in range(_EP - 1):
        h, c, x = _TC_STEP(ea, query, h, c, wihT, whhT, bih, bhh,
                           f1T, f1b, f2T, f2b)
        current, ea = _SAMPLE_STEP(
            embeddings, adj_nodes, adj_etypes, edge_embeds, current, x,
            noise[step])
        out_embeds.append(ea[:, _D:])
        out_inds.append(current)
    return (jnp.stack(out_embeds, axis=0), jnp.stack(out_inds, axis=0),
            start_embeds, end_embeds)
